# baseline (speedup 1.0000x reference)
import functools

import jax
import jax.numpy as jnp
from jax import lax
from jax.experimental import pallas as pl
from jax.experimental.pallas import tpu as pltpu
from jax.experimental.pallas import tpu_sc as plsc

N = 16384
C = 1000
L = 16
NC = 2
NS = 16
NW = NC * NS
ROWS_PER_W = N // NW
CHUNK = 128
NCHUNK = ROWS_PER_W // CHUNK

_mesh = plsc.VectorSubcoreMesh(core_axis_name="c", subcore_axis_name="s")


@functools.partial(
    pl.kernel,
    mesh=_mesh,
    out_type=jax.ShapeDtypeStruct((NW, L), jnp.float32),
    scratch_types=[
        pltpu.VMEM((ROWS_PER_W,), jnp.int32),
        pltpu.VMEM((ROWS_PER_W,), jnp.float32),
        pltpu.VMEM((ROWS_PER_W,), jnp.int32),
        pltpu.VMEM((ROWS_PER_W,), jnp.float32),
        pltpu.VMEM((L,), jnp.float32),
        pltpu.SemaphoreType.DMA,
        pltpu.SemaphoreType.DMA,
        pltpu.SemaphoreType.DMA,
        pltpu.SemaphoreType.DMA,
        pltpu.SemaphoreType.DMA,
    ],
)
def _gan_loss_sc(prob_flat_hbm, tgt_hbm, rwd_hbm, out_hbm,
                 tgt_v, rwd_v, idx_v, sel_v, part_v, sem_in,
                 sg0, sg1, sg2, sg3):
    sem_g = (sg0, sg1, sg2, sg3)
    sid = lax.axis_index("s")
    cid = lax.axis_index("c")
    wid = sid * NC + cid
    base = wid * ROWS_PER_W

    cp_t = [
        pltpu.async_copy(
            tgt_hbm.at[pl.ds(base + k * CHUNK, CHUNK)],
            tgt_v.at[pl.ds(k * CHUNK, CHUNK)],
            sem_g[k],
        )
        for k in range(NCHUNK)
    ]
    cp_r = pltpu.async_copy(rwd_hbm.at[pl.ds(base, ROWS_PER_W)], rwd_v, sem_in)

    lane = lax.broadcasted_iota(jnp.int32, (L,), 0)
    gathers = []
    for k in range(NCHUNK):
        cp_t[k].wait()

        def idx_body(j, _, k=k):
            off = k * CHUNK + j * L
            s = ((base + off) >> 7) * 1024 + ((base + off) & 127)
            t = tgt_v[pl.ds(off, L)]
            idx_v[pl.ds(off, L)] = (
                ((t >> 3) << 17) + ((t & 7) << 7) + (s + lane)
            )
            return 0
        lax.fori_loop(0, CHUNK // L, idx_body, 0, unroll=2)
        gathers.append(
            pltpu.async_copy(
                prob_flat_hbm.at[idx_v.at[pl.ds(k * CHUNK, CHUNK)]],
                sel_v.at[pl.ds(k * CHUNK, CHUNK)],
                sem_g[k],
            )
        )

    cp_r.wait()
    one = jnp.full((L,), 1.0 + 1e-6, jnp.float32)
    acc = jnp.zeros((L,), jnp.float32)
    for k in range(NCHUNK):
        gathers[k].wait()

        def acc_body(j, acc, k=k):
            off = k * CHUNK + j * L
            return acc + sel_v[pl.ds(off, L)] * (one - rwd_v[pl.ds(off, L)])

        acc = lax.fori_loop(0, CHUNK // L, acc_body, acc, unroll=4)
    part_v[...] = acc
    pltpu.sync_copy(part_v, out_hbm.at[wid])


def kernel(prob, target, reward):
    tgt = target.astype(jnp.int32)
    rwd = reward.astype(jnp.float32)
    prob_phys = prob.reshape(128, 128, 125, 8).transpose(2, 0, 3, 1).reshape(-1)
    partials = _gan_loss_sc(prob_phys, tgt, rwd)
    return jnp.sum(partials) * (-1.0 / N)


# speedup vs baseline: 1.0037x; 1.0037x over previous
import functools

import jax
import jax.numpy as jnp
from jax import lax
from jax.experimental import pallas as pl
from jax.experimental.pallas import tpu as pltpu
from jax.experimental.pallas import tpu_sc as plsc

N = 16384
C = 1000
L = 16
NC = 2
NS = 16
NW = NC * NS
ROWS_PER_W = N // NW
CHUNK = 128
NCHUNK = ROWS_PER_W // CHUNK

_mesh = plsc.VectorSubcoreMesh(core_axis_name="c", subcore_axis_name="s")


@functools.partial(
    pl.kernel,
    mesh=_mesh,
    out_type=jax.ShapeDtypeStruct((NW, L), jnp.float32),
    scratch_types=[
        pltpu.VMEM((ROWS_PER_W,), jnp.int32),
        pltpu.VMEM((ROWS_PER_W,), jnp.float32),
        pltpu.VMEM((ROWS_PER_W,), jnp.int32),
        pltpu.VMEM((ROWS_PER_W,), jnp.float32),
        pltpu.VMEM((L,), jnp.float32),
        pltpu.SemaphoreType.DMA,
        pltpu.SemaphoreType.DMA,
        pltpu.SemaphoreType.DMA,
        pltpu.SemaphoreType.DMA,
        pltpu.SemaphoreType.DMA,
    ],
)
def _gan_loss_sc(prob_flat_hbm, tgt_hbm, rwd_hbm, out_hbm,
                 tgt_v, rwd_v, idx_v, sel_v, part_v, sem_in,
                 sg0, sg1, sg2, sg3):
    sem_g = (sg0, sg1, sg2, sg3)
    sid = lax.axis_index("s")
    cid = lax.axis_index("c")
    wid = sid * NC + cid
    base = wid * ROWS_PER_W

    cp_t = [
        pltpu.async_copy(
            tgt_hbm.at[pl.ds(base + k * CHUNK, CHUNK)],
            tgt_v.at[pl.ds(k * CHUNK, CHUNK)],
            sem_g[k],
        )
        for k in range(NCHUNK)
    ]
    cp_r = pltpu.async_copy(rwd_hbm.at[pl.ds(base, ROWS_PER_W)], rwd_v, sem_in)

    lane = lax.broadcasted_iota(jnp.int32, (L,), 0)
    gathers = []
    for k in range(NCHUNK):
        cp_t[k].wait()

        def idx_body(j, _, k=k):
            off = k * CHUNK + j * L
            s = ((base + off) >> 7) * 1024 + ((base + off) & 127)
            t = tgt_v[pl.ds(off, L)]
            idx_v[pl.ds(off, L)] = (
                ((t >> 3) << 17) + ((t & 7) << 7) + (s + lane)
            )
            return 0
        lax.fori_loop(0, CHUNK // L, idx_body, 0, unroll=4)
        gathers.append(
            pltpu.async_copy(
                prob_flat_hbm.at[idx_v.at[pl.ds(k * CHUNK, CHUNK)]],
                sel_v.at[pl.ds(k * CHUNK, CHUNK)],
                sem_g[k],
            )
        )

    cp_r.wait()
    one = jnp.full((L,), 1.0 + 1e-6, jnp.float32)
    acc = jnp.zeros((L,), jnp.float32)
    for k in range(NCHUNK):
        gathers[k].wait()

        def acc_body(j, acc, k=k):
            off = k * CHUNK + j * L
            return acc + sel_v[pl.ds(off, L)] * (one - rwd_v[pl.ds(off, L)])

        acc = lax.fori_loop(0, CHUNK // L, acc_body, acc, unroll=8)
    part_v[...] = acc
    pltpu.sync_copy(part_v, out_hbm.at[wid])


def kernel(prob, target, reward):
    tgt = target.astype(jnp.int32)
    rwd = reward.astype(jnp.float32)
    prob_phys = prob.reshape(128, 128, 125, 8).transpose(2, 0, 3, 1).reshape(-1)
    partials = _gan_loss_sc(prob_phys, tgt, rwd)
    return jnp.sum(partials) * (-1.0 / N)
